# baseline (speedup 1.0000x reference)
import functools

import jax
import jax.numpy as jnp
from jax import lax
from jax.experimental import pallas as pl
from jax.experimental.pallas import tpu as pltpu
from jax.experimental.pallas import tpu_sc as plsc

EMBED_DIM = 64
BATCH = 16384

_NUM_CORES = 2
_NUM_SUBCORES = 16
_NUM_WORKERS = _NUM_CORES * _NUM_SUBCORES
_B_PER_W = BATCH // _NUM_WORKERS
_GROUP = 16
_N_GROUPS = _B_PER_W // _GROUP

_mesh = plsc.VectorSubcoreMesh(core_axis_name="c", subcore_axis_name="s")


@functools.partial(
    pl.kernel,
    out_type=(
        jax.ShapeDtypeStruct((BATCH, EMBED_DIM), jnp.float32),
        jax.ShapeDtypeStruct((BATCH, EMBED_DIM), jnp.float32),
    ),
    mesh=_mesh,
    scratch_types=[
        pltpu.VMEM((_B_PER_W,), jnp.int32),
        pltpu.VMEM((_B_PER_W,), jnp.int32),
        pltpu.SemaphoreType.DMA,
    ],
)
def _lookup_kernel(ent_hbm, rel_hbm, eids_hbm, rids_hbm, out_e, out_r,
                   idx_e, idx_r, sem):
    wid = lax.axis_index("s") * _NUM_CORES + lax.axis_index("c")
    base = wid * _B_PER_W

    pltpu.sync_copy(eids_hbm.at[pl.ds(base, _B_PER_W)], idx_e)
    pltpu.sync_copy(rids_hbm.at[pl.ds(base, _B_PER_W)], idx_r)

    def do_group(g, _):
        evals = idx_e[pl.ds(g * _GROUP, _GROUP)]
        rvals = idx_r[pl.ds(g * _GROUP, _GROUP)]
        for j in range(_GROUP):
            k = g * _GROUP + j
            pltpu.async_copy(ent_hbm.at[evals[j]], out_e.at[base + k], sem)
            pltpu.async_copy(rel_hbm.at[rvals[j]], out_r.at[base + k], sem)
        return 0

    lax.fori_loop(0, _N_GROUPS, do_group, 0)

    pltpu.make_async_copy(ent_hbm.at[pl.ds(0, _B_PER_W)],
                          out_e.at[pl.ds(base, _B_PER_W)], sem).wait()
    pltpu.make_async_copy(rel_hbm.at[pl.ds(0, _B_PER_W)],
                          out_r.at[pl.ds(base, _B_PER_W)], sem).wait()


def kernel(entity_embedding, relation_embedding, entity_ids, relation_ids):
    return _lookup_kernel(entity_embedding, relation_embedding,
                          entity_ids.astype(jnp.int32),
                          relation_ids.astype(jnp.int32))


# speedup vs baseline: 1.9739x; 1.9739x over previous
import functools

import jax
import jax.numpy as jnp
from jax import lax
from jax.experimental import pallas as pl
from jax.experimental.pallas import tpu as pltpu
from jax.experimental.pallas import tpu_sc as plsc

EMBED_DIM = 64
BATCH = 16384

_NUM_CORES = 2
_NUM_SUBCORES = 16
_NUM_WORKERS = _NUM_CORES * _NUM_SUBCORES
_B_PER_W = BATCH // _NUM_WORKERS
_GROUP = 16
_N_GROUPS = _B_PER_W // _GROUP
_W_ELEMS = _B_PER_W * EMBED_DIM

_mesh = plsc.VectorSubcoreMesh(core_axis_name="c", subcore_axis_name="s")


@functools.partial(
    pl.kernel,
    out_type=(
        jax.ShapeDtypeStruct((BATCH * EMBED_DIM,), jnp.float32),
        jax.ShapeDtypeStruct((BATCH * EMBED_DIM,), jnp.float32),
    ),
    mesh=_mesh,
    scratch_types=[
        pltpu.VMEM((_B_PER_W,), jnp.int32),
        pltpu.VMEM((_GROUP, EMBED_DIM), jnp.float32),
        pltpu.VMEM((_W_ELEMS,), jnp.float32),
        pltpu.SemaphoreType.DMA,
    ],
)
def _lookup_kernel(ent_hbm, rel_hbm, eids_hbm, rids_hbm, out_e, out_r,
                   idx_v, rows2d, rows1d, sem):
    wid = lax.axis_index("s") * _NUM_CORES + lax.axis_index("c")
    base = wid * _B_PER_W

    def one_table(table_hbm, ids_hbm, out_1d):
        pltpu.sync_copy(ids_hbm.at[pl.ds(base, _B_PER_W)], idx_v)

        def fetch_group(g, _):
            vals = idx_v[pl.ds(g * _GROUP, _GROUP)]
            copies = [
                pltpu.async_copy(table_hbm.at[vals[j]], rows2d.at[j], sem)
                for j in range(_GROUP)
            ]
            for cp in copies:
                cp.wait()
            for j in range(_GROUP):
                off = (g * _GROUP + j) * EMBED_DIM
                for c in range(EMBED_DIM // 16):
                    rows1d[pl.ds(off + c * 16, 16)] = (
                        rows2d[j, pl.ds(c * 16, 16)])
            return 0

        lax.fori_loop(0, _N_GROUPS, fetch_group, 0)
        pltpu.sync_copy(rows1d, out_1d.at[pl.ds(base * EMBED_DIM, _W_ELEMS)])

    one_table(ent_hbm, eids_hbm, out_e)
    one_table(rel_hbm, rids_hbm, out_r)


def kernel(entity_embedding, relation_embedding, entity_ids, relation_ids):
    flat_e, flat_r = _lookup_kernel(entity_embedding, relation_embedding,
                                    entity_ids.astype(jnp.int32),
                                    relation_ids.astype(jnp.int32))
    return (flat_e.reshape(BATCH, EMBED_DIM), flat_r.reshape(BATCH, EMBED_DIM))
